# Initial kernel scaffold; baseline (speedup 1.0000x reference)
import jax
import jax.numpy as jnp
from jax.experimental import pallas as pl


def kernel(input_ids, token_type_ids, word_mask, word_emb, pos_emb, type_emb, arr_emb, gamma, beta):
    raise NotImplementedError("write your pallas kernel here")


# speedup vs baseline: 2.2455x; 2.2455x over previous
import functools

import jax
import jax.numpy as jnp
from jax import lax
from jax.experimental import pallas as pl
from jax.experimental.pallas import tpu as pltpu
from jax.experimental.pallas import tpu_sc as plsc

VOCAB = 100000
HIDDEN = 128
SEQ = 2048
BATCH = 4
EPS = 1e-12
L = 16
NC = 2
NS = 16
NW = NC * NS
NTOK = BATCH * SEQ
TPW = NTOK // NW
GROUPS = TPW // L
CHUNKS = HIDDEN // L
GCHUNK = 128


_DN = lax.GatherDimensionNumbers(
    offset_dims=(), collapsed_slice_dims=(0,), start_index_map=(0,))


def _gather16(v, idx):
    return lax.gather(v, idx[:, None], _DN, slice_sizes=(1,),
                      mode=lax.GatherScatterMode.PROMISE_IN_BOUNDS)


def _lane_bcast(v, j):
    return _gather16(v, jnp.full((L,), j, dtype=jnp.int32))


def _allsum(v):
    iota = jnp.arange(L, dtype=jnp.int32)
    for k in (1, 2, 4, 8):
        v = v + _gather16(v, iota ^ k)
    return v


_MESH = plsc.VectorSubcoreMesh(core_axis_name="c", subcore_axis_name="s")


@functools.partial(
    pl.kernel,
    out_type=jax.ShapeDtypeStruct((NTOK, HIDDEN), jnp.float32),
    mesh=_MESH,
    scratch_types=[
        pltpu.VMEM((TPW // GCHUNK, GCHUNK), jnp.int32),
        pltpu.VMEM((TPW, HIDDEN), jnp.float32),
        pltpu.VMEM((TPW, HIDDEN), jnp.float32),
        pltpu.VMEM((TPW,), jnp.int32),
        pltpu.VMEM((TPW,), jnp.int32),
        pltpu.VMEM((5, HIDDEN), jnp.float32),
        pltpu.SemaphoreType.DMA,
    ],
)
def _emb_ln_kernel(word_hbm, ids_hbm, tt_hbm, wm_hbm, pos_hbm, small_hbm,
                   out_hbm, idx_v, w_v, p_v, tt_v, wm_v, small_v, sem):
    cid = lax.axis_index("c")
    sid = lax.axis_index("s")
    wid = sid * NC + cid

    pltpu.sync_copy(ids_hbm.at[wid], idx_v)
    cps = [
        pltpu.async_copy(word_hbm.at[idx_v.at[k]],
                         w_v.at[pl.ds(k * GCHUNK, GCHUNK)], sem)
        for k in range(TPW // GCHUNK)
    ]
    pbase = (wid * TPW) % SEQ
    pltpu.sync_copy(pos_hbm.at[pl.ds(pbase, TPW)], p_v)
    pltpu.sync_copy(tt_hbm.at[wid], tt_v)
    pltpu.sync_copy(wm_hbm.at[wid], wm_v)
    pltpu.sync_copy(small_hbm, small_v)
    for cp in cps:
        cp.wait()

    base_c = [small_v[0, pl.ds(c * L, L)] for c in range(CHUNKS)]
    dt_c = [small_v[1, pl.ds(c * L, L)] for c in range(CHUNKS)]
    da_c = [small_v[2, pl.ds(c * L, L)] for c in range(CHUNKS)]
    g_c = [small_v[3, pl.ds(c * L, L)] for c in range(CHUNKS)]
    b_c = [small_v[4, pl.ds(c * L, L)] for c in range(CHUNKS)]
    def group_body(g, carry):
        base = pl.multiple_of(g * L, L)
        ttf = tt_v[pl.ds(base, L)].astype(jnp.float32)
        wmf = wm_v[pl.ds(base, L)].astype(jnp.float32)

        for j in range(L):
            t = base + j
            ttj = _lane_bcast(ttf, j)
            wmj = _lane_bcast(wmf, j)
            xs = []
            sv = None
            qv = None
            for c in range(CHUNKS):
                sl = pl.ds(c * L, L)
                x = w_v[t, sl] + p_v[t, sl] + base_c[c]
                x = x + ttj * dt_c[c] + wmj * da_c[c]
                xs.append(x)
                sv = x if sv is None else sv + x
                qv = x * x if qv is None else qv + x * x
            s_all = _allsum(sv)
            q_all = _allsum(qv)
            mean = s_all * (1.0 / HIDDEN)
            var = q_all * (1.0 / HIDDEN) - mean * mean
            a = var + EPS
            ai = lax.bitcast_convert_type(a, jnp.int32)
            y = lax.bitcast_convert_type(
                jnp.int32(0x5F3759DF) - lax.shift_right_arithmetic(ai, 1),
                jnp.float32)
            for _ in range(3):
                y = y * (1.5 - 0.5 * a * y * y)
            rstd = y
            for c in range(CHUNKS):
                sl = pl.ds(c * L, L)
                w_v[t, sl] = (xs[c] - mean) * rstd * g_c[c] + b_c[c]
        return carry

    lax.fori_loop(0, GROUPS, group_body, 0)

    pltpu.sync_copy(w_v, out_hbm.at[pl.ds(wid * TPW, TPW)])


def kernel(input_ids, token_type_ids, word_mask, word_emb, pos_emb,
           type_emb, arr_emb, gamma, beta):
    ids3 = input_ids.reshape(NW, TPW // GCHUNK, GCHUNK).astype(jnp.int32)
    tt2 = token_type_ids.reshape(NW, TPW).astype(jnp.int32)
    wm2 = word_mask.reshape(NW, TPW).astype(jnp.int32)
    small = jnp.stack([
        type_emb[0] + arr_emb[0],
        type_emb[1] - type_emb[0],
        arr_emb[1] - arr_emb[0],
        gamma,
        beta,
    ])
    out = _emb_ln_kernel(word_emb, ids3, tt2, wm2, pos_emb, small)
    return out.reshape(BATCH, SEQ, HIDDEN)
